# baseline (speedup 1.0000x reference)
import functools

import jax
import jax.numpy as jnp
from jax import lax
from jax.experimental import pallas as pl
from jax.experimental.pallas import tpu as pltpu
from jax.experimental.pallas import tpu_sc as plsc

B = 16
TP = 2500
TW = 625
W1 = TW + 1
H = 128
E = 160000
N = B * TW
K = 6
NSTEPS = 5
BW = B * W1
NK = N * K

NC = 2
NS = 16
CW = 128
CH = 128

f32 = jnp.float32
i32 = jnp.int32

_MESH = dict(core_axis_name="c", subcore_axis_name="s", num_cores=NC,
             num_subcores=NS)


def _fill(ref, nrows, width, val):
    v = jnp.full((16,), val, f32)

    def body(i, _):
        for j in range(width // 16):
            ref[i, pl.ds(j * 16, 16)] = v
        return 0

    lax.fori_loop(0, nrows, body, 0)


def _wid():
    return lax.axis_index("s") * NC + lax.axis_index("c")


_RS = 624


def _zero_shared(sid, acc_s, zrows_v, total):
    base = sid * _RS
    for j in range(4):
        pltpu.sync_copy(zrows_v, acc_s.at[pl.ds(base + j * CH, CH)])
    pltpu.sync_copy(zrows_v.at[pl.ds(0, _RS - 4 * CH)],
                    acc_s.at[pl.ds(base + 4 * CH, _RS - 4 * CH)])
    rem = total - NS * _RS
    if rem:
        @pl.when(sid == 0)
        def _():
            pltpu.sync_copy(zrows_v.at[pl.ds(0, rem)],
                            acc_s.at[pl.ds(NS * _RS, rem)])


def _writeback(cid, sid, acc_s, out, total):
    base = sid * _RS
    pltpu.sync_copy(acc_s.at[pl.ds(base, _RS)], out.at[cid, pl.ds(base, _RS)])
    rem = total - NS * _RS
    if rem:
        @pl.when(sid == 0)
        def _():
            pltpu.sync_copy(acc_s.at[pl.ds(NS * _RS, rem)],
                            out.at[cid, pl.ds(NS * _RS, rem)])


HB = BW // NC
CHP = 80
_RS2 = 312


def _pool_body(x_hbm, idx_hbm, hsum_out, cnt_out,
               idx_v, rows_v, ones_v, acc_s, cnt_s, sem):
    cid = lax.axis_index("c")
    sid = lax.axis_index("s")
    _fill(rows_v, CHP, H, 0.0)
    _fill(ones_v, CHP, CW, 1.0)
    zbase = sid * _RS2
    for j in range(4):
        nz = CHP if j < 3 else _RS2 - 3 * CHP
        pltpu.sync_copy(rows_v.at[pl.ds(0, nz)],
                        acc_s.at[pl.ds(zbase + j * CHP, nz)])
        pltpu.sync_copy(rows_v.at[pl.ds(0, nz)],
                        cnt_s.at[pl.ds(zbase + j * CHP, nz)])

    @pl.when(sid == 0)
    def _():
        pltpu.sync_copy(rows_v.at[pl.ds(0, HB - NS * _RS2)],
                        acc_s.at[pl.ds(NS * _RS2, HB - NS * _RS2)])
        pltpu.sync_copy(rows_v.at[pl.ds(0, HB - NS * _RS2)],
                        cnt_s.at[pl.ds(NS * _RS2, HB - NS * _RS2)])

    plsc.subcore_barrier()

    jcnt = (249 - sid) // NS + 1

    def chunk_body(j, _):
        off = (cid * 250 + sid + NS * j) * CHP
        pltpu.sync_copy(idx_hbm.at[pl.ds(off, CHP)], idx_v)
        pltpu.sync_copy(x_hbm.at[pl.ds(off, CHP)], rows_v)
        pltpu.sync_copy(rows_v, acc_s.at[idx_v], add=True)
        pltpu.sync_copy(ones_v, cnt_s.at[idx_v], add=True)
        return 0

    lax.fori_loop(0, jcnt, chunk_body, 0)
    plsc.subcore_barrier()

    base = sid * _RS2
    pltpu.sync_copy(acc_s.at[pl.ds(base, _RS2)],
                    hsum_out.at[pl.ds(cid * HB + base, _RS2)])
    pltpu.sync_copy(cnt_s.at[pl.ds(base, _RS2)],
                    cnt_out.at[pl.ds(cid * HB + base, _RS2)])

    @pl.when(sid == 0)
    def _():
        pltpu.sync_copy(acc_s.at[pl.ds(NS * _RS2, HB - NS * _RS2)],
                        hsum_out.at[pl.ds(cid * HB + NS * _RS2,
                                          HB - NS * _RS2)])
        pltpu.sync_copy(cnt_s.at[pl.ds(NS * _RS2, HB - NS * _RS2)],
                        cnt_out.at[pl.ds(cid * HB + NS * _RS2,
                                         HB - NS * _RS2)])


def _pool(x, flat_idx_mod):
    kfn = pl.kernel(
        _pool_body,
        out_type=(jax.ShapeDtypeStruct((BW, H), f32),
                  jax.ShapeDtypeStruct((BW, CW), f32)),
        mesh=plsc.VectorSubcoreMesh(**_MESH),
        scratch_types=[
            pltpu.VMEM((CHP,), i32),
            pltpu.VMEM((CHP, H), f32),
            pltpu.VMEM((CHP, CW), f32),
            pltpu.VMEM_SHARED((HB, H), f32),
            pltpu.VMEM_SHARED((HB, CW), f32),
            pltpu.SemaphoreType.DMA,
        ],
    )
    return kfn(x, flat_idx_mod)


EPT = 40
E2 = NC * NS * EPT * CH
ECH = E2 // CH


_KB = 2


def _edge_body(table_hbm, gidx_hbm, dst_hbm, parts_out,
               gall_v, dall_v, r0, r1, acc_s, sem):
    cid = lax.axis_index("c")
    sid = lax.axis_index("s")
    wid = _wid()
    rows = (r0, r1)
    _fill(r0, CH, H, 0.0)
    _zero_shared(sid, acc_s, r0, N + 8)
    pltpu.sync_copy(gidx_hbm.at[pl.ds(wid * EPT, EPT)], gall_v)
    pltpu.sync_copy(dst_hbm.at[pl.ds(wid * EPT, EPT)], dall_v)
    plsc.subcore_barrier()

    def chunk_body(j, _):
        base = _KB * j
        hs = [pltpu.async_copy(table_hbm.at[gall_v.at[base + b]], rows[b],
                               sem) for b in range(_KB)]
        for b in range(_KB):
            hs[b].wait()
        for b in range(_KB):
            pltpu.sync_copy(rows[b], acc_s.at[dall_v.at[base + b]], add=True)
        return 0

    lax.fori_loop(0, EPT // _KB, chunk_body, 0)
    plsc.subcore_barrier()
    _writeback(cid, sid, acc_s, parts_out, N)


def _edge(table, gidx2d, dst2d):
    kfn = pl.kernel(
        _edge_body,
        out_type=jax.ShapeDtypeStruct((NC, N, H), f32),
        mesh=plsc.VectorSubcoreMesh(**_MESH),
        scratch_types=[
            pltpu.VMEM((EPT, CH), i32),
            pltpu.VMEM((EPT, CH), i32),
            pltpu.VMEM((CH, H), f32),
            pltpu.VMEM((CH, H), f32),
            pltpu.VMEM_SHARED((N + 8, H), f32),
            pltpu.SemaphoreType.DMA,
        ],
    )
    return kfn(table, gidx2d, dst2d)


def _fgather_body(table_hbm, idx_hbm, out_hbm, idx_v, idxt_v, rows_v, sem):
    wid = _wid()
    jfull = (311 - wid) // 32 + 1

    def chunk_body(j, _):
        off = (wid + 32 * j) * CH
        pltpu.sync_copy(idx_hbm.at[pl.ds(off, CH)], idx_v)
        pltpu.async_copy(table_hbm.at[idx_v], rows_v, sem).wait()
        pltpu.sync_copy(rows_v, out_hbm.at[pl.ds(off, CH)])
        return 0

    lax.fori_loop(0, jfull, chunk_body, 0)

    @pl.when(wid == 312 % 32)
    def _():
        off = 312 * CH
        pltpu.sync_copy(idx_hbm.at[pl.ds(off, 64)], idxt_v)
        pltpu.async_copy(table_hbm.at[idxt_v], rows_v.at[pl.ds(0, 64)],
                         sem).wait()
        pltpu.sync_copy(rows_v.at[pl.ds(0, 64)], out_hbm.at[pl.ds(off, 64)])


def _fgather(table, flat_idx):
    kfn = pl.kernel(
        _fgather_body,
        out_type=jax.ShapeDtypeStruct((B * TP, H), f32),
        mesh=plsc.VectorSubcoreMesh(**_MESH),
        scratch_types=[
            pltpu.VMEM((CH,), i32),
            pltpu.VMEM((64,), i32),
            pltpu.VMEM((CH, H), f32),
            pltpu.SemaphoreType.DMA,
        ],
    )
    return kfn(table, flat_idx)


_R = 2000


def _norm_kernel(hs_ref, cnt_ref, out_ref):
    hs = hs_ref[0]
    c = cnt_ref[0]
    cc = jnp.clip(c[1:, 0:1], 1.0, None)
    out_ref[0] = hs[1:, :] / cc


def _normalize(hsum, cnt):
    return pl.pallas_call(
        _norm_kernel,
        grid=(B,),
        in_specs=[
            pl.BlockSpec((1, W1, H), lambda b: (b, 0, 0)),
            pl.BlockSpec((1, W1, CW), lambda b: (b, 0, 0)),
        ],
        out_specs=pl.BlockSpec((1, TW, H), lambda b: (b, 0, 0)),
        out_shape=jax.ShapeDtypeStruct((B, TW, H), f32),
    )(hsum.reshape(B, W1, H), cnt.reshape(B, W1, CW))


def _hw_kernel(h_ref, lw_ref, lb_ref, hw_ref):
    hw_ref[...] = lax.dot_general(
        h_ref[...], lw_ref[...], (((1,), (1,)), ((), ())),
        preferred_element_type=f32) + lb_ref[...]


def _hw_only(h, lwc, lbc):
    return pl.pallas_call(
        _hw_kernel,
        grid=(N // _R,),
        in_specs=[
            pl.BlockSpec((_R, H), lambda i: (i, 0)),
            pl.BlockSpec((K * H, H), lambda i: (0, 0)),
            pl.BlockSpec((1, K * H), lambda i: (0, 0)),
        ],
        out_specs=pl.BlockSpec((_R, K * H), lambda i: (i, 0)),
        out_shape=jax.ShapeDtypeStruct((N, K * H), f32),
    )(h, lwc, lbc)


def _gru_kernel(with_hw, h_ref, parts_ref, wih_ref, whh_ref, bih_ref,
                bhh_ref, lw_ref, lb_ref, hn_ref, *hw_ref):
    a = parts_ref[0] + parts_ref[1]
    h = h_ref[...]
    gi = lax.dot_general(a, wih_ref[...], (((1,), (1,)), ((), ())),
                         preferred_element_type=f32) + bih_ref[...]
    gh = lax.dot_general(h, whh_ref[...], (((1,), (1,)), ((), ())),
                         preferred_element_type=f32) + bhh_ref[...]
    r = jax.nn.sigmoid(gi[:, :H] + gh[:, :H])
    z = jax.nn.sigmoid(gi[:, H:2 * H] + gh[:, H:2 * H])
    n = jnp.tanh(gi[:, 2 * H:] + r * gh[:, 2 * H:])
    hn = (1.0 - z) * n + z * h
    hn_ref[...] = hn
    if with_hw:
        hw_ref[0][...] = lax.dot_general(
            hn, lw_ref[...], (((1,), (1,)), ((), ())),
            preferred_element_type=f32) + lb_ref[...]


def _gru(h, parts, wih, whh, bih, bhh, lwc, lbc, with_hw):
    outs = [jax.ShapeDtypeStruct((N, H), f32)]
    out_specs = [pl.BlockSpec((_R, H), lambda i: (i, 0))]
    if with_hw:
        outs.append(jax.ShapeDtypeStruct((N, K * H), f32))
        out_specs.append(pl.BlockSpec((_R, K * H), lambda i: (i, 0)))
    res = pl.pallas_call(
        functools.partial(_gru_kernel, with_hw),
        grid=(N // _R,),
        in_specs=[
            pl.BlockSpec((_R, H), lambda i: (i, 0)),
            pl.BlockSpec((NC, _R, H), lambda i: (0, i, 0)),
            pl.BlockSpec((3 * H, H), lambda i: (0, 0)),
            pl.BlockSpec((3 * H, H), lambda i: (0, 0)),
            pl.BlockSpec((1, 3 * H), lambda i: (0, 0)),
            pl.BlockSpec((1, 3 * H), lambda i: (0, 0)),
            pl.BlockSpec((K * H, H), lambda i: (0, 0)),
            pl.BlockSpec((1, K * H), lambda i: (0, 0)),
        ],
        out_specs=out_specs,
        out_shape=outs,
    )(h, parts, wih, whh, bih, bhh, lwc, lbc)
    return res if with_hw else (res[0], None)


def _skip_kernel(i_ref, g1_ref, g2_ref, out_ref):
    s = i_ref[0] + g1_ref[0] + g2_ref[0]
    out_ref[0] = jnp.concatenate([jnp.zeros((1, H), f32), s], axis=0)


def _skip_pad(inp, g1, g2):
    return pl.pallas_call(
        _skip_kernel,
        grid=(B,),
        in_specs=[pl.BlockSpec((1, TW, H), lambda b: (b, 0, 0))] * 3,
        out_specs=pl.BlockSpec((1, W1, H), lambda b: (b, 0, 0)),
        out_shape=jax.ShapeDtypeStruct((B, W1, H), f32),
    )(inp.reshape(B, TW, H), g1.reshape(B, TW, H), g2.reshape(B, TW, H))


def kernel(ph_encoding, ph2word, edge_index, etypes,
           ggc1_linW, ggc1_linb, ggc1_Wih, ggc1_Whh, ggc1_bih, ggc1_bhh,
           ggc2_linW, ggc2_linb, ggc2_Wih, ggc2_Whh, ggc2_bih, ggc2_bhh):
    x = jnp.transpose(ph_encoding, (0, 2, 1)).reshape(B * TP, H)
    flat_idx = (jnp.arange(B, dtype=i32)[:, None] * W1
                + ph2word.astype(i32)).reshape(-1)
    gidx = (edge_index[0].astype(i32) * K + etypes.astype(i32))
    dst = edge_index[1].astype(i32)
    gidx2d = jnp.concatenate(
        [gidx, jnp.zeros((E2 - E,), i32)]).reshape(ECH, CH)
    dst2d = jnp.concatenate(
        [dst, jnp.full((E2 - E,), N, i32)]).reshape(ECH, CH)

    lw1 = ggc1_linW.reshape(K * H, H)
    lb1 = ggc1_linb.reshape(1, K * H)
    lw2 = ggc2_linW.reshape(K * H, H)
    lb2 = ggc2_linb.reshape(1, K * H)
    b1ih = ggc1_bih.reshape(1, 3 * H)
    b1hh = ggc1_bhh.reshape(1, 3 * H)
    b2ih = ggc2_bih.reshape(1, 3 * H)
    b2hh = ggc2_bhh.reshape(1, 3 * H)

    hsum, cnt = _pool(x, flat_idx % HB)
    inp = _normalize(hsum, cnt).reshape(N, H)

    hw = _hw_only(inp, lw1, lb1)
    h = inp
    g1 = None
    for layer in (1, 2):
        wih, whh, bih, bhh = ((ggc1_Wih, ggc1_Whh, b1ih, b1hh) if layer == 1
                              else (ggc2_Wih, ggc2_Whh, b2ih, b2hh))
        for step in range(NSTEPS):
            parts = _edge(hw.reshape(NK, H), gidx2d, dst2d)
            last = (layer == 2 and step == NSTEPS - 1)
            nlw, nlb = (lw1, lb1) if (layer == 1 and step < NSTEPS - 1) \
                else (lw2, lb2)
            h, hw = _gru(h, parts, wih, whh, bih, bhh, nlw, nlb,
                         with_hw=not last)
        if layer == 1:
            g1 = h

    padded = _skip_pad(inp, g1, h)
    out_rows = _fgather(padded.reshape(BW, H), flat_idx)
    return jnp.transpose(out_rows.reshape(B, TP, H), (0, 2, 1))


# speedup vs baseline: 1.8117x; 1.8117x over previous
import functools

import jax
import jax.numpy as jnp
from jax import lax
from jax.experimental import pallas as pl
from jax.experimental.pallas import tpu as pltpu
from jax.experimental.pallas import tpu_sc as plsc

B = 16
TP = 2500
TW = 625
W1 = TW + 1
H = 128
E = 160000
N = B * TW
K = 6
NSTEPS = 5
BW = B * W1
NK = N * K

NC = 2
NS = 16
CW = 128
CH = 128

f32 = jnp.float32
i32 = jnp.int32

_MESH = dict(core_axis_name="c", subcore_axis_name="s", num_cores=NC,
             num_subcores=NS)


def _fill(ref, nrows, width, val):
    v = jnp.full((16,), val, f32)

    def body(i, _):
        for j in range(width // 16):
            ref[i, pl.ds(j * 16, 16)] = v
        return 0

    lax.fori_loop(0, nrows, body, 0)


def _wid():
    return lax.axis_index("s") * NC + lax.axis_index("c")


_RS = 624


def _zero_shared(sid, acc_s, zrows_v, total):
    base = sid * _RS
    for j in range(4):
        pltpu.sync_copy(zrows_v, acc_s.at[pl.ds(base + j * CH, CH)])
    pltpu.sync_copy(zrows_v.at[pl.ds(0, _RS - 4 * CH)],
                    acc_s.at[pl.ds(base + 4 * CH, _RS - 4 * CH)])
    rem = total - NS * _RS
    if rem:
        @pl.when(sid == 0)
        def _():
            pltpu.sync_copy(zrows_v.at[pl.ds(0, rem)],
                            acc_s.at[pl.ds(NS * _RS, rem)])


def _writeback(cid, sid, acc_s, out, total):
    base = sid * _RS
    pltpu.sync_copy(acc_s.at[pl.ds(base, _RS)], out.at[cid, pl.ds(base, _RS)])
    rem = total - NS * _RS
    if rem:
        @pl.when(sid == 0)
        def _():
            pltpu.sync_copy(acc_s.at[pl.ds(NS * _RS, rem)],
                            out.at[cid, pl.ds(NS * _RS, rem)])


HB = BW // NC
CHP = 80
_RS2 = 312


def _pool_body(x_hbm, idx_hbm, hsum_out, cnt_out,
               idx_v, rows_v, ones_v, acc_s, cnt_s, sem):
    cid = lax.axis_index("c")
    sid = lax.axis_index("s")
    _fill(rows_v, CHP, H, 0.0)
    _fill(ones_v, CHP, CW, 1.0)
    zbase = sid * _RS2
    for j in range(4):
        nz = CHP if j < 3 else _RS2 - 3 * CHP
        pltpu.sync_copy(rows_v.at[pl.ds(0, nz)],
                        acc_s.at[pl.ds(zbase + j * CHP, nz)])
        pltpu.sync_copy(rows_v.at[pl.ds(0, nz)],
                        cnt_s.at[pl.ds(zbase + j * CHP, nz)])

    @pl.when(sid == 0)
    def _():
        pltpu.sync_copy(rows_v.at[pl.ds(0, HB - NS * _RS2)],
                        acc_s.at[pl.ds(NS * _RS2, HB - NS * _RS2)])
        pltpu.sync_copy(rows_v.at[pl.ds(0, HB - NS * _RS2)],
                        cnt_s.at[pl.ds(NS * _RS2, HB - NS * _RS2)])

    plsc.subcore_barrier()

    jcnt = (249 - sid) // NS + 1

    def chunk_body(j, _):
        off = (cid * 250 + sid + NS * j) * CHP
        pltpu.sync_copy(idx_hbm.at[pl.ds(off, CHP)], idx_v)
        pltpu.sync_copy(x_hbm.at[pl.ds(off, CHP)], rows_v)
        pltpu.sync_copy(rows_v, acc_s.at[idx_v], add=True)
        pltpu.sync_copy(ones_v, cnt_s.at[idx_v], add=True)
        return 0

    lax.fori_loop(0, jcnt, chunk_body, 0)
    plsc.subcore_barrier()

    base = sid * _RS2
    pltpu.sync_copy(acc_s.at[pl.ds(base, _RS2)],
                    hsum_out.at[pl.ds(cid * HB + base, _RS2)])
    pltpu.sync_copy(cnt_s.at[pl.ds(base, _RS2)],
                    cnt_out.at[pl.ds(cid * HB + base, _RS2)])

    @pl.when(sid == 0)
    def _():
        pltpu.sync_copy(acc_s.at[pl.ds(NS * _RS2, HB - NS * _RS2)],
                        hsum_out.at[pl.ds(cid * HB + NS * _RS2,
                                          HB - NS * _RS2)])
        pltpu.sync_copy(cnt_s.at[pl.ds(NS * _RS2, HB - NS * _RS2)],
                        cnt_out.at[pl.ds(cid * HB + NS * _RS2,
                                         HB - NS * _RS2)])


def _pool(x, flat_idx_mod):
    kfn = pl.kernel(
        _pool_body,
        out_type=(jax.ShapeDtypeStruct((BW, H), f32),
                  jax.ShapeDtypeStruct((BW, CW), f32)),
        mesh=plsc.VectorSubcoreMesh(**_MESH),
        scratch_types=[
            pltpu.VMEM((CHP,), i32),
            pltpu.VMEM((CHP, H), f32),
            pltpu.VMEM((CHP, CW), f32),
            pltpu.VMEM_SHARED((HB, H), f32),
            pltpu.VMEM_SHARED((HB, CW), f32),
            pltpu.SemaphoreType.DMA,
        ],
    )
    return kfn(x, flat_idx_mod)


def _edge_body(table_hbm, gidx_hbm, dst_hbm, parts_out,
               idxg_v, idxd_v, rows_v, acc_s, sem):
    cid = lax.axis_index("c")
    sid = lax.axis_index("s")
    wid = _wid()
    _fill(rows_v, CH, H, 0.0)
    _zero_shared(sid, acc_s, rows_v, N)
    plsc.subcore_barrier()

    jcnt = (1249 - wid) // 32 + 1

    def chunk_body(j, _):
        off = (wid + 32 * j) * CH
        pltpu.sync_copy(gidx_hbm.at[pl.ds(off, CH)], idxg_v)
        pltpu.sync_copy(dst_hbm.at[pl.ds(off, CH)], idxd_v)
        pltpu.async_copy(table_hbm.at[idxg_v], rows_v, sem).wait()
        pltpu.sync_copy(rows_v, acc_s.at[idxd_v], add=True)
        return 0

    lax.fori_loop(0, jcnt, chunk_body, 0)
    plsc.subcore_barrier()
    _writeback(cid, sid, acc_s, parts_out, N)


def _edge(table, gidx, dst):
    kfn = pl.kernel(
        _edge_body,
        out_type=jax.ShapeDtypeStruct((NC, N, H), f32),
        mesh=plsc.VectorSubcoreMesh(**_MESH),
        scratch_types=[
            pltpu.VMEM((CH,), i32),
            pltpu.VMEM((CH,), i32),
            pltpu.VMEM((CH, H), f32),
            pltpu.VMEM_SHARED((N, H), f32),
            pltpu.SemaphoreType.DMA,
        ],
    )
    return kfn(table, gidx, dst)


def _fgather_body(table_hbm, idx_hbm, out_hbm, idx_v, idxt_v, rows_v, sem):
    wid = _wid()
    jfull = (311 - wid) // 32 + 1

    def chunk_body(j, _):
        off = (wid + 32 * j) * CH
        pltpu.sync_copy(idx_hbm.at[pl.ds(off, CH)], idx_v)
        pltpu.async_copy(table_hbm.at[idx_v], rows_v, sem).wait()
        pltpu.sync_copy(rows_v, out_hbm.at[pl.ds(off, CH)])
        return 0

    lax.fori_loop(0, jfull, chunk_body, 0)

    @pl.when(wid == 312 % 32)
    def _():
        off = 312 * CH
        pltpu.sync_copy(idx_hbm.at[pl.ds(off, 64)], idxt_v)
        pltpu.async_copy(table_hbm.at[idxt_v], rows_v.at[pl.ds(0, 64)],
                         sem).wait()
        pltpu.sync_copy(rows_v.at[pl.ds(0, 64)], out_hbm.at[pl.ds(off, 64)])


def _fgather(table, flat_idx):
    kfn = pl.kernel(
        _fgather_body,
        out_type=jax.ShapeDtypeStruct((B * TP, H), f32),
        mesh=plsc.VectorSubcoreMesh(**_MESH),
        scratch_types=[
            pltpu.VMEM((CH,), i32),
            pltpu.VMEM((64,), i32),
            pltpu.VMEM((CH, H), f32),
            pltpu.SemaphoreType.DMA,
        ],
    )
    return kfn(table, flat_idx)


_R = 2000


def _norm_hw_kernel(hs_ref, cnt_ref, lw_ref, lb_ref, inp_ref, hw_ref):
    hs = hs_ref[0]
    c = cnt_ref[0]
    cc = jnp.clip(c[1:, 0:1], 1.0, None)
    inp = hs[1:, :] / cc
    inp_ref[0] = inp
    hw_ref[0] = lax.dot_general(
        inp, lw_ref[...], (((1,), (1,)), ((), ())),
        preferred_element_type=f32) + lb_ref[...]


def _normalize_hw(hsum, cnt, lwc, lbc):
    return pl.pallas_call(
        _norm_hw_kernel,
        grid=(B,),
        in_specs=[
            pl.BlockSpec((1, W1, H), lambda b: (b, 0, 0)),
            pl.BlockSpec((1, W1, CW), lambda b: (b, 0, 0)),
            pl.BlockSpec((K * H, H), lambda b: (0, 0)),
            pl.BlockSpec((1, K * H), lambda b: (0, 0)),
        ],
        out_specs=[
            pl.BlockSpec((1, TW, H), lambda b: (b, 0, 0)),
            pl.BlockSpec((1, TW, K * H), lambda b: (b, 0, 0)),
        ],
        out_shape=[jax.ShapeDtypeStruct((B, TW, H), f32),
                   jax.ShapeDtypeStruct((B, TW, K * H), f32)],
    )(hsum.reshape(B, W1, H), cnt.reshape(B, W1, CW), lwc, lbc)


def _gru_kernel(with_hw, h_ref, parts_ref, wih_ref, whh_ref, bih_ref,
                bhh_ref, lw_ref, lb_ref, hn_ref, *hw_ref):
    a = parts_ref[0] + parts_ref[1]
    h = h_ref[...]
    gi = lax.dot_general(a, wih_ref[...], (((1,), (1,)), ((), ())),
                         preferred_element_type=f32) + bih_ref[...]
    gh = lax.dot_general(h, whh_ref[...], (((1,), (1,)), ((), ())),
                         preferred_element_type=f32) + bhh_ref[...]
    r = jax.nn.sigmoid(gi[:, :H] + gh[:, :H])
    z = jax.nn.sigmoid(gi[:, H:2 * H] + gh[:, H:2 * H])
    n = jnp.tanh(gi[:, 2 * H:] + r * gh[:, 2 * H:])
    hn = (1.0 - z) * n + z * h
    hn_ref[...] = hn
    if with_hw:
        hw_ref[0][...] = lax.dot_general(
            hn, lw_ref[...], (((1,), (1,)), ((), ())),
            preferred_element_type=f32) + lb_ref[...]


def _gru(h, parts, wih, whh, bih, bhh, lwc, lbc, with_hw):
    outs = [jax.ShapeDtypeStruct((N, H), f32)]
    out_specs = [pl.BlockSpec((_R, H), lambda i: (i, 0))]
    if with_hw:
        outs.append(jax.ShapeDtypeStruct((N, K * H), f32))
        out_specs.append(pl.BlockSpec((_R, K * H), lambda i: (i, 0)))
    res = pl.pallas_call(
        functools.partial(_gru_kernel, with_hw),
        grid=(N // _R,),
        in_specs=[
            pl.BlockSpec((_R, H), lambda i: (i, 0)),
            pl.BlockSpec((NC, _R, H), lambda i: (0, i, 0)),
            pl.BlockSpec((3 * H, H), lambda i: (0, 0)),
            pl.BlockSpec((3 * H, H), lambda i: (0, 0)),
            pl.BlockSpec((1, 3 * H), lambda i: (0, 0)),
            pl.BlockSpec((1, 3 * H), lambda i: (0, 0)),
            pl.BlockSpec((K * H, H), lambda i: (0, 0)),
            pl.BlockSpec((1, K * H), lambda i: (0, 0)),
        ],
        out_specs=out_specs,
        out_shape=outs,
    )(h, parts, wih, whh, bih, bhh, lwc, lbc)
    return res if with_hw else (res[0], None)


def _gru_skip_kernel(h_ref, parts_ref, wih_ref, whh_ref, bih_ref, bhh_ref,
                     i_ref, g1_ref, out_ref):
    a = parts_ref[0, 0] + parts_ref[1, 0]
    h = h_ref[0]
    gi = lax.dot_general(a, wih_ref[...], (((1,), (1,)), ((), ())),
                         preferred_element_type=f32) + bih_ref[...]
    gh = lax.dot_general(h, whh_ref[...], (((1,), (1,)), ((), ())),
                         preferred_element_type=f32) + bhh_ref[...]
    r = jax.nn.sigmoid(gi[:, :H] + gh[:, :H])
    z = jax.nn.sigmoid(gi[:, H:2 * H] + gh[:, H:2 * H])
    n = jnp.tanh(gi[:, 2 * H:] + r * gh[:, 2 * H:])
    hn = (1.0 - z) * n + z * h
    s = i_ref[0] + g1_ref[0] + hn
    out_ref[0] = jnp.concatenate([jnp.zeros((1, H), f32), s], axis=0)


def _gru_skip_pad(h, parts, wih, whh, bih, bhh, inp, g1):
    return pl.pallas_call(
        _gru_skip_kernel,
        grid=(B,),
        in_specs=[
            pl.BlockSpec((1, TW, H), lambda b: (b, 0, 0)),
            pl.BlockSpec((NC, 1, TW, H), lambda b: (0, b, 0, 0)),
            pl.BlockSpec((3 * H, H), lambda b: (0, 0)),
            pl.BlockSpec((3 * H, H), lambda b: (0, 0)),
            pl.BlockSpec((1, 3 * H), lambda b: (0, 0)),
            pl.BlockSpec((1, 3 * H), lambda b: (0, 0)),
            pl.BlockSpec((1, TW, H), lambda b: (b, 0, 0)),
            pl.BlockSpec((1, TW, H), lambda b: (b, 0, 0)),
        ],
        out_specs=pl.BlockSpec((1, W1, H), lambda b: (b, 0, 0)),
        out_shape=jax.ShapeDtypeStruct((B, W1, H), f32),
    )(h.reshape(B, TW, H), parts.reshape(NC, B, TW, H), wih, whh, bih, bhh,
      inp.reshape(B, TW, H), g1.reshape(B, TW, H))


def kernel(ph_encoding, ph2word, edge_index, etypes,
           ggc1_linW, ggc1_linb, ggc1_Wih, ggc1_Whh, ggc1_bih, ggc1_bhh,
           ggc2_linW, ggc2_linb, ggc2_Wih, ggc2_Whh, ggc2_bih, ggc2_bhh):
    x = jnp.transpose(ph_encoding, (0, 2, 1)).reshape(B * TP, H)
    flat_idx = (jnp.arange(B, dtype=i32)[:, None] * W1
                + ph2word.astype(i32)).reshape(-1)
    gidx = (edge_index[0].astype(i32) * K + etypes.astype(i32))
    dst = edge_index[1].astype(i32)

    lw1 = ggc1_linW.reshape(K * H, H)
    lb1 = ggc1_linb.reshape(1, K * H)
    lw2 = ggc2_linW.reshape(K * H, H)
    lb2 = ggc2_linb.reshape(1, K * H)
    b1ih = ggc1_bih.reshape(1, 3 * H)
    b1hh = ggc1_bhh.reshape(1, 3 * H)
    b2ih = ggc2_bih.reshape(1, 3 * H)
    b2hh = ggc2_bhh.reshape(1, 3 * H)

    hsum, cnt = _pool(x, flat_idx % HB)
    inp3, hw3 = _normalize_hw(hsum, cnt, lw1, lb1)
    inp = inp3.reshape(N, H)
    hw = hw3.reshape(N, K * H)
    h = inp
    g1 = None
    padded = None
    for layer in (1, 2):
        wih, whh, bih, bhh = ((ggc1_Wih, ggc1_Whh, b1ih, b1hh) if layer == 1
                              else (ggc2_Wih, ggc2_Whh, b2ih, b2hh))
        for step in range(NSTEPS):
            parts = _edge(hw.reshape(NK, H), gidx, dst)
            if layer == 2 and step == NSTEPS - 1:
                padded = _gru_skip_pad(h, parts, wih, whh, bih, bhh, inp, g1)
            else:
                nlw, nlb = (lw1, lb1) if (layer == 1 and step < NSTEPS - 1) \
                    else (lw2, lb2)
                h, hw = _gru(h, parts, wih, whh, bih, bhh, nlw, nlb,
                             with_hw=True)
        if layer == 1:
            g1 = h

    out_rows = _fgather(padded.reshape(BW, H), flat_idx)
    return jnp.transpose(out_rows.reshape(B, TP, H), (0, 2, 1))
